# baseline (speedup 1.0000x reference)
import functools
import math

import jax
import jax.numpy as jnp
from jax import lax
from jax.experimental import pallas as pl
from jax.experimental.pallas import tpu as pltpu
from jax.experimental.pallas import tpu_sc as plsc

_D = 64
_B = 4096
_L = 200
_V = 1000000
_N = _B * _L
_NC = 2
_NS = 16
_NW = _NC * _NS
_LANES = 16
_SCALE = math.sqrt(_D)

_VT = _V // 128
_VMAIN = _VT * 128
_T_PER_W = _VT // _NW
_T_EXTRA = _VT - _T_PER_W * _NW

_LPB = 2
_NBLK = _L // _LPB


def _iota16():
    return lax.iota(jnp.int32, 16)


def _k1_body(wt, wtail, tbl, in_v, out_v, tail_v, sem0, sem1):
    wid = lax.axis_index("s") * _NC + lax.axis_index("c")
    start = wid * _T_PER_W + jnp.minimum(wid, _T_EXTRA)
    cnt = _T_PER_W + jnp.where(wid < _T_EXTRA, 1, 0)
    sems = (sem0, sem1)

    qidx = []
    for g in range(8):
        q = g * 16 + _iota16()
        qidx.append((q % 64, q // 64))

    def issue(j, sem, b):
        pltpu.async_copy(wt.at[:, pl.ds(j * 128, 128)], in_v.at[b], sem)

    issue(start, sems[0], 0)

    @pl.loop(0, (_T_PER_W + 2) // 2 + 1)
    def _pair(p):
        for b in range(2):
            k = p * 2 + b

            @pl.when(k < cnt)
            def _(b=b, k=k):
                j = start + k
                pltpu.make_async_copy(
                    wt.at[:, pl.ds(j * 128, 128)], in_v.at[b], sems[b]
                ).wait()

                @pl.when(k + 1 < cnt)
                def _():
                    issue(j + 1, sems[1 - b], 1 - b)

                src = in_v.at[b]
                for g in range(8):
                    d0, d1 = qidx[g]

                    @plsc.parallel_loop(0, 64, 1, unroll=4)
                    def _tp(q, _g=g, _d0=d0, _d1=d1, _src=src, _b=b):
                        vals = plsc.load_gather(_src, [_d0, _d1 + 2 * q])
                        out_v[_b, q, pl.ds(_g * 16, 16)] = vals * _SCALE

                pltpu.sync_copy(out_v.at[b], tbl.at[pl.ds(j * 64, 64)])

    @pl.when(wid == _NW - 1)
    def _tail():
        pltpu.sync_copy(wtail, tail_v)
        for g in range(8):
            d0, d1 = qidx[g]

            @plsc.parallel_loop(0, 32, 1, unroll=4)
            def _tp(p, _g=g, _d0=d0, _d1=d1):
                vals = plsc.load_gather(tail_v, [_d0, _d1 + 2 * p])
                out_v[0, p, pl.ds(_g * 16, 16)] = vals * _SCALE

        pltpu.sync_copy(out_v.at[0, pl.ds(0, 32)], tbl.at[pl.ds(_VMAIN // 2, 32)])


@functools.partial(
    pl.kernel,
    out_type=jax.ShapeDtypeStruct((_V // 2, 128), jnp.float32),
    mesh=plsc.VectorSubcoreMesh(core_axis_name="c", subcore_axis_name="s"),
    scratch_types=[
        pltpu.VMEM((2, _D, 128), jnp.float32),
        pltpu.VMEM((2, 64, 128), jnp.float32),
        pltpu.VMEM((_D, _D), jnp.float32),
        pltpu.SemaphoreType.DMA,
        pltpu.SemaphoreType.DMA,
    ],
    compiler_params=pltpu.CompilerParams(use_tc_tiling_on_sc=True, needs_layout_passes=False),
)
def _repack(wt, wtail, tbl, in_v, out_v, tail_v, sem0, sem1):
    _k1_body(wt, wtail, tbl, in_v, out_v, tail_v, sem0, sem1)


def _k2_body(tbl, idxt, out, idx_v, rows_v, out_v, sem0, sem1, semw):
    wid = lax.axis_index("s") * _NC + lax.axis_index("c")
    sems = (sem0, sem1)

    rowsel = [g * 16 + _iota16() for g in range(8)]

    def issue(blk, b):
        l0 = blk * _LPB
        pltpu.sync_copy(
            idxt.at[pl.ds(l0, _LPB), pl.ds(wid * 128, 128)], idx_v.at[b]
        )
        for u in range(_LPB):
            pltpu.async_copy(
                tbl.at[idx_v.at[b, u]],
                rows_v.at[b, pl.ds(u * 128, 128)],
                sems[b],
            )

    def drain_writes(blk, b):
        l0 = blk * _LPB
        for u in range(_LPB):
            for ci in range(8):
                pltpu.make_async_copy(
                    out_v.at[b, u, ci], out.at[l0 + u, ci, wid], semw
                ).wait()

    issue(jnp.int32(0), 0)

    @pl.loop(0, _NBLK // 2)
    def _pair(p):
        for b in range(2):
            blk = p * 2 + b

            for u in range(_LPB):
                pltpu.make_async_copy(
                    tbl.at[idx_v.at[b, u]],
                    rows_v.at[b, pl.ds(u * 128, 128)],
                    sems[b],
                ).wait()

            @pl.when(blk + 1 < _NBLK)
            def _(b=b, blk=blk):
                issue(blk + 1, 1 - b)

            @pl.when(blk >= 2)
            def _(b=b, blk=blk):
                drain_writes(blk - 2, b)

            for u in range(_LPB):
                src = rows_v.at[b, pl.ds(u * 128, 128)]
                for g in range(8):
                    rs = rowsel[g]

                    @plsc.parallel_loop(0, _D, 1, unroll=4)
                    def _tp(c, _g=g, _rs=rs, _u=u, _src=src, _b=b):
                        vals = plsc.load_gather(
                            _src, [_rs, jnp.zeros((16,), jnp.int32) + c]
                        )
                        ci = lax.shift_right_logical(c, 3)
                        c8 = lax.rem(c, 8)
                        out_v[_b, _u, ci, c8, pl.ds(_g * 16, 16)] = vals

            l0 = blk * _LPB
            for u in range(_LPB):
                for ci in range(8):
                    pltpu.async_copy(
                        out_v.at[b, u, ci], out.at[l0 + u, ci, wid], semw
                    )

    @pl.loop(0, 1)
    def _fin(_):
        for blkf in (_NBLK - 2, _NBLK - 1):
            drain_writes(jnp.int32(blkf), blkf % 2)


@functools.partial(
    pl.kernel,
    out_type=jax.ShapeDtypeStruct((_L, 8, _NW, 8, 128), jnp.float32),
    mesh=plsc.VectorSubcoreMesh(core_axis_name="c", subcore_axis_name="s"),
    scratch_types=[
        pltpu.VMEM((2, _LPB, 128), jnp.int32),
        pltpu.VMEM((2, _LPB * 128, _D), jnp.float32),
        pltpu.VMEM((2, _LPB, 8, 8, 128), jnp.float32),
        pltpu.SemaphoreType.DMA,
        pltpu.SemaphoreType.DMA,
        pltpu.SemaphoreType.DMA,
    ],
    compiler_params=pltpu.CompilerParams(use_tc_tiling_on_sc=False, needs_layout_passes=False),
)
def _gather(tbl, idxt, out, idx_v, rows_v, out_v, sem0, sem1, semw):
    _k2_body(tbl, idxt, out, idx_v, rows_v, out_v, sem0, sem1, semw)


def kernel(x, embedding_weight):
    wt = embedding_weight.T
    wtail = embedding_weight[_VMAIN:].T
    tbl2 = _repack(wt, wtail)
    tbl = tbl2.reshape(_V, _D)
    idxt = x.astype(jnp.int32).T
    o5 = _gather(tbl, idxt)
    return o5.transpose(2, 4, 0, 1, 3).reshape(_B, _L, _D)
